# Initial kernel scaffold; baseline (speedup 1.0000x reference)
import jax
import jax.numpy as jnp
from jax.experimental import pallas as pl


def kernel(idx, targets, table):
    raise NotImplementedError("write your pallas kernel here")


# speedup vs baseline: 1.3924x; 1.3924x over previous
import functools

import jax
import jax.numpy as jnp
from jax import lax
from jax.experimental import pallas as pl
from jax.experimental.pallas import tpu as pltpu
from jax.experimental.pallas import tpu_sc as plsc

C = 1000
N = 51200
NC, NS, L = 2, 16, 16
NW = NC * NS
PER_W = N // NW
GW = 32
SGW = 64


def _vector_mesh():
    return plsc.VectorSubcoreMesh(core_axis_name="c", subcore_axis_name="s")


def _rowlz_body(t_ref, o_ref):
    x = t_ref[...]
    m = jnp.max(x, axis=1, keepdims=True)
    s = jnp.sum(jnp.exp(x - m), axis=1, keepdims=True)
    o_ref[...] = m + jnp.log(s)


def _row_logsumexp(table):
    return pl.pallas_call(
        _rowlz_body,
        out_shape=jax.ShapeDtypeStruct((C, 1), jnp.float32),
    )(table)


def _gather_rows(table, idx_flat):
    n_chunks = PER_W // GW

    @functools.partial(
        pl.kernel,
        out_type=jax.ShapeDtypeStruct((N, C), jnp.float32),
        mesh=_vector_mesh(),
        compiler_params=pltpu.CompilerParams(use_tc_tiling_on_sc=False),
        scratch_types=[
            pltpu.VMEM((PER_W,), jnp.int32),
            pltpu.VMEM((GW, C), jnp.float32),
            pltpu.SemaphoreType.DMA,
            pltpu.SemaphoreType.DMA,
        ],
    )
    def k(table_hbm, idx_hbm, out_hbm, idx_v, rows0, g0, s0):
        wid = lax.axis_index("s") * NC + lax.axis_index("c")
        base = wid * PER_W
        pltpu.sync_copy(idx_hbm.at[pl.ds(base, PER_W)], idx_v)

        @pl.loop(0, n_chunks)
        def _(c):
            pltpu.async_copy(
                table_hbm.at[idx_v.at[pl.ds(c * GW, GW)]], rows0, g0).wait()
            pltpu.async_copy(
                rows0, out_hbm.at[pl.ds(base + c * GW, GW)], s0).wait()

    return k(table, idx_flat)


def _loss_parts(table_flat, rowlz_flat, idx_flat, tgt_flat):
    @functools.partial(
        pl.kernel,
        out_type=jax.ShapeDtypeStruct((NW, L), jnp.float32),
        mesh=_vector_mesh(),
        scratch_types=[
            pltpu.VMEM((PER_W,), jnp.int32),
            pltpu.VMEM((PER_W,), jnp.int32),
            pltpu.VMEM((PER_W,), jnp.float32),
            pltpu.VMEM((PER_W,), jnp.float32),
            pltpu.VMEM((L,), jnp.float32),
        ],
    )
    def k(tab_hbm, lz_hbm, idx_hbm, tgt_hbm, out_hbm,
          idx_v, tgt_v, lz_v, pk_v, acc_v):
        wid = lax.axis_index("s") * NC + lax.axis_index("c")
        base = wid * PER_W
        pltpu.sync_copy(idx_hbm.at[pl.ds(base, PER_W)], idx_v)
        pltpu.sync_copy(tgt_hbm.at[pl.ds(base, PER_W)], tgt_v)

        @pl.loop(0, PER_W, step=L)
        def _(j):
            sl = pl.ds(j, L)
            tgt_v[sl] = idx_v[sl] * C + tgt_v[sl]

        @pl.loop(0, PER_W, step=SGW)
        def _(j):
            sl = pl.ds(j, SGW)
            pltpu.sync_copy(lz_hbm.at[idx_v.at[sl]], lz_v.at[sl])
            pltpu.sync_copy(tab_hbm.at[tgt_v.at[sl]], pk_v.at[sl])

        acc_v[...] = jnp.zeros((L,), jnp.float32)

        @pl.loop(0, PER_W, step=L)
        def _(j):
            sl = pl.ds(j, L)
            acc_v[...] = acc_v[...] + (lz_v[sl] - pk_v[sl])

        pltpu.sync_copy(acc_v, out_hbm.at[wid])

    return k(table_flat, rowlz_flat, idx_flat, tgt_flat)


def _reduce_body(p_ref, o_ref):
    o_ref[...] = (jnp.sum(p_ref[...]) / N).reshape(1, 1)


def _reduce_loss(parts):
    return pl.pallas_call(
        _reduce_body,
        out_shape=jax.ShapeDtypeStruct((1, 1), jnp.float32),
    )(parts)


def kernel(idx, targets, table):
    idx_flat = idx.reshape(-1).astype(jnp.int32)
    tgt_flat = targets.reshape(-1).astype(jnp.int32)
    logits2 = _gather_rows(table, idx_flat)
    rowlz = _row_logsumexp(table)
    parts = _loss_parts(table.reshape(-1), rowlz.reshape(-1), idx_flat, tgt_flat)
    loss = _reduce_loss(parts)
    return (logits2, loss[0, 0])


# speedup vs baseline: 1.4558x; 1.0455x over previous
import functools

import jax
import jax.numpy as jnp
from jax import lax
from jax.experimental import pallas as pl
from jax.experimental.pallas import tpu as pltpu
from jax.experimental.pallas import tpu_sc as plsc

C = 1000
N = 51200
NC, NS, L = 2, 16, 16
NW = NC * NS
PER_W = N // NW
GW = 40
SGW = 64


def _vector_mesh():
    return plsc.VectorSubcoreMesh(core_axis_name="c", subcore_axis_name="s")


def _rowlz_body(t_ref, o_ref):
    x = t_ref[...]
    m = jnp.max(x, axis=1, keepdims=True)
    s = jnp.sum(jnp.exp(x - m), axis=1, keepdims=True)
    o_ref[...] = m + jnp.log(s)


def _row_logsumexp(table):
    return pl.pallas_call(
        _rowlz_body,
        out_shape=jax.ShapeDtypeStruct((C, 1), jnp.float32),
    )(table)


def _gather_rows(table, idx_flat):
    n_chunks = PER_W // GW

    @functools.partial(
        pl.kernel,
        out_type=jax.ShapeDtypeStruct((N, C), jnp.float32),
        mesh=_vector_mesh(),
        compiler_params=pltpu.CompilerParams(use_tc_tiling_on_sc=False),
        scratch_types=[
            pltpu.VMEM((PER_W,), jnp.int32),
            pltpu.VMEM((GW, C), jnp.float32),
            pltpu.VMEM((GW, C), jnp.float32),
            pltpu.SemaphoreType.DMA,
            pltpu.SemaphoreType.DMA,
            pltpu.SemaphoreType.DMA,
            pltpu.SemaphoreType.DMA,
        ],
    )
    def k(table_hbm, idx_hbm, out_hbm, idx_v, rows0, rows1, g0, g1, s0, s1):
        wid = lax.axis_index("s") * NC + lax.axis_index("c")
        base = wid * PER_W
        pltpu.sync_copy(idx_hbm.at[pl.ds(base, PER_W)], idx_v)

        rows = (rows0, rows1)
        gsem = (g0, g1)
        ssem = (s0, s1)

        def gather_start(c, b):
            pltpu.make_async_copy(
                table_hbm.at[idx_v.at[pl.ds(c * GW, GW)]], rows[b], gsem[b]
            ).start()

        def gather_wait(c, b):
            pltpu.make_async_copy(
                table_hbm.at[idx_v.at[pl.ds(c * GW, GW)]], rows[b], gsem[b]
            ).wait()

        def write_start(c, b):
            pltpu.make_async_copy(
                rows[b], out_hbm.at[pl.ds(base + c * GW, GW)], ssem[b]
            ).start()

        def write_wait(c, b):
            pltpu.make_async_copy(
                rows[b], out_hbm.at[pl.ds(base + c * GW, GW)], ssem[b]
            ).wait()

        gather_start(0, 0)
        gather_start(1, 1)

        @pl.loop(0, n_chunks, step=2)
        def _(c):
            for b in range(2):
                gather_wait(c + b, b)
                write_start(c + b, b)
            for b in range(2):
                nxt = c + 2 + b

                @pl.when(nxt < n_chunks)
                def _():
                    write_wait(c + b, b)
                    gather_start(nxt, b)

        write_wait(n_chunks - 2, 0)
        write_wait(n_chunks - 1, 1)

    return k(table, idx_flat)


def _loss_parts(table_flat, rowlz_flat, idx_flat, tgt_flat):
    @functools.partial(
        pl.kernel,
        out_type=jax.ShapeDtypeStruct((NW, L), jnp.float32),
        mesh=_vector_mesh(),
        scratch_types=[
            pltpu.VMEM((PER_W,), jnp.int32),
            pltpu.VMEM((PER_W,), jnp.int32),
            pltpu.VMEM((PER_W,), jnp.float32),
            pltpu.VMEM((PER_W,), jnp.float32),
            pltpu.VMEM((L,), jnp.float32),
        ],
    )
    def k(tab_hbm, lz_hbm, idx_hbm, tgt_hbm, out_hbm,
          idx_v, tgt_v, lz_v, pk_v, acc_v):
        wid = lax.axis_index("s") * NC + lax.axis_index("c")
        base = wid * PER_W
        pltpu.sync_copy(idx_hbm.at[pl.ds(base, PER_W)], idx_v)
        pltpu.sync_copy(tgt_hbm.at[pl.ds(base, PER_W)], tgt_v)

        @pl.loop(0, PER_W, step=L)
        def _(j):
            sl = pl.ds(j, L)
            tgt_v[sl] = idx_v[sl] * C + tgt_v[sl]

        @pl.loop(0, PER_W, step=SGW)
        def _(j):
            sl = pl.ds(j, SGW)
            pltpu.sync_copy(lz_hbm.at[idx_v.at[sl]], lz_v.at[sl])
            pltpu.sync_copy(tab_hbm.at[tgt_v.at[sl]], pk_v.at[sl])

        acc_v[...] = jnp.zeros((L,), jnp.float32)

        @pl.loop(0, PER_W, step=L)
        def _(j):
            sl = pl.ds(j, L)
            acc_v[...] = acc_v[...] + (lz_v[sl] - pk_v[sl])

        pltpu.sync_copy(acc_v, out_hbm.at[wid])

    return k(table_flat, rowlz_flat, idx_flat, tgt_flat)


def _reduce_body(p_ref, o_ref):
    o_ref[...] = (jnp.sum(p_ref[...]) / N).reshape(1, 1)


def _reduce_loss(parts):
    return pl.pallas_call(
        _reduce_body,
        out_shape=jax.ShapeDtypeStruct((1, 1), jnp.float32),
    )(parts)


def kernel(idx, targets, table):
    idx_flat = idx.reshape(-1).astype(jnp.int32)
    tgt_flat = targets.reshape(-1).astype(jnp.int32)
    logits2 = _gather_rows(table, idx_flat)
    rowlz = _row_logsumexp(table)
    parts = _loss_parts(table.reshape(-1), rowlz.reshape(-1), idx_flat, tgt_flat)
    loss = _reduce_loss(parts)
    return (logits2, loss[0, 0])
